# baseline (speedup 1.0000x reference)
import jax
import jax.numpy as jnp
from jax import lax
from jax.experimental import pallas as pl
from jax.experimental.pallas import tpu as pltpu
from jax.experimental.pallas import tpu_sc as plsc

D = 768
F = 3072
E = 8
T = 2048
K = 2
NPAIR = T * K
B = 128
NB = 40
PADDED = NB * B
NW = 32
CHUNK = 128


def _gate_body(x_ref, wg_ref, bg_ref, wp_ref, pos_ref, gid_ref):
    x = x_ref[...]
    scores = jnp.dot(x, wg_ref[...], preferred_element_type=jnp.float32)
    scores = scores + bg_ref[...]
    io = lax.broadcasted_iota(jnp.int32, (T, E), 1).astype(jnp.float32)
    m1 = jnp.max(scores, axis=1, keepdims=True)
    a1 = jnp.min(jnp.where(scores == m1, io, float(E)), axis=1, keepdims=True)
    masked = jnp.where(io == a1, -jnp.inf, scores)
    m2 = jnp.max(masked, axis=1, keepdims=True)
    a2 = jnp.min(jnp.where(masked == m2, io, float(E)), axis=1, keepdims=True)
    w1 = 1.0 / (1.0 + jnp.exp(m2 - m1))
    w2 = 1.0 - w1
    wp_ref[...] = jnp.concatenate([w1, w2], axis=1)

    oh1 = (io == a1).astype(jnp.float32)
    oh2 = (io == a2).astype(jnp.float32)
    tot1 = jnp.sum(oh1, axis=0, keepdims=True)
    tot = tot1 + jnp.sum(oh2, axis=0, keepdims=True)
    pu = jnp.floor((tot + float(B - 1)) * (1.0 / B)) * float(B)
    e0 = lax.broadcasted_iota(jnp.int32, (E, E), 0).astype(jnp.float32)
    e1 = lax.broadcasted_iota(jnp.int32, (E, E), 1).astype(jnp.float32)
    triE = (e0 < e1).astype(jnp.float32)
    base = jnp.dot(pu, triE, preferred_element_type=jnp.float32)

    t0 = lax.broadcasted_iota(jnp.int32, (T, T), 0)
    t1 = lax.broadcasted_iota(jnp.int32, (T, T), 1)
    triT = (t1 < t0).astype(jnp.bfloat16)
    oh_cat = jnp.concatenate([oh1, oh2], axis=1).astype(jnp.bfloat16)
    pref = jnp.dot(triT, oh_cat, preferred_element_type=jnp.float32)
    pos0 = jnp.sum(oh1 * (base + pref[:, :E]), axis=1, keepdims=True)
    pos1 = jnp.sum(oh2 * (base + tot1 + pref[:, E:]), axis=1, keepdims=True)
    pos_ref[...] = jnp.concatenate([pos0, pos1], axis=1).astype(jnp.int32)

    tp = jnp.sum(pu)
    bs = lax.broadcasted_iota(jnp.int32, (48, E), 0).astype(jnp.float32) * float(B)
    pend = base + pu
    gid = jnp.sum((bs >= pend).astype(jnp.float32), axis=1, keepdims=True)
    gid = jnp.minimum(gid, float(E - 1))
    valid = (bs < tp).astype(jnp.float32)
    gid_last = jnp.max(jnp.where(valid > 0.0, jnp.broadcast_to(gid, (48, E)),
                                 0.0))
    gidf = jnp.where(valid > 0.0, jnp.broadcast_to(gid, (48, E)), gid_last)
    gid_ref[...] = gidf.astype(jnp.int32)


def _gate(x, Wg, bg2):
    out_shapes = (
        jax.ShapeDtypeStruct((T, K), jnp.float32),
        jax.ShapeDtypeStruct((T, K), jnp.int32),
        jax.ShapeDtypeStruct((48, E), jnp.int32),
    )
    return pl.pallas_call(_gate_body, out_shape=out_shapes)(x, Wg, bg2)


def _dispatch_body(pos_hbm, x_hbm, xs_hbm, posbuf0, posbuf1, rows, sem):
    wid = lax.axis_index("s") * 2 + lax.axis_index("c")
    base_j = pl.multiple_of(wid * CHUNK, CHUNK)
    tok0 = pl.multiple_of(base_j & (T - 1), CHUNK)
    half = CHUNK // 2
    pltpu.sync_copy(pos_hbm.at[pl.ds(base_j, half)], posbuf0)
    pltpu.sync_copy(pos_hbm.at[pl.ds(base_j + half, half)], posbuf1)
    pltpu.sync_copy(x_hbm.at[pl.ds(tok0, half)], rows)
    pltpu.async_copy(rows, xs_hbm.at[posbuf0], sem).wait()
    pltpu.sync_copy(x_hbm.at[pl.ds(tok0 + half, half)], rows)
    pltpu.async_copy(rows, xs_hbm.at[posbuf1], sem).wait()


def _dispatch(pos_flat, x):
    mesh = plsc.VectorSubcoreMesh(core_axis_name="c", subcore_axis_name="s")
    fn = pl.kernel(
        _dispatch_body,
        out_type=jax.ShapeDtypeStruct((PADDED, D), jnp.float32),
        mesh=mesh,
        scratch_types=[
            pltpu.VMEM((CHUNK // 2,), jnp.int32),
            pltpu.VMEM((CHUNK // 2,), jnp.int32),
            pltpu.VMEM((CHUNK // 2, D), jnp.float32),
            pltpu.SemaphoreType.DMA,
        ],
    )
    return fn(pos_flat, x)


def _ffn_body(gid_ref, pos0_ref, pos1_ref, x_ref, w1_ref, b1_ref, w2_ref,
              b2_ref, out_ref):
    row0 = pl.program_id(0) * B
    slot = row0 + lax.broadcasted_iota(jnp.int32, (B, T), 0)
    oh = ((pos0_ref[...] == slot).astype(jnp.float32) +
          (pos1_ref[...] == slot).astype(jnp.float32))
    xv = jnp.dot(oh, x_ref[...], preferred_element_type=jnp.float32)
    h = jnp.dot(xv, w1_ref[0], preferred_element_type=jnp.float32)
    h = jnp.maximum(h + b1_ref[0], 0.0)
    y = jnp.dot(h, w2_ref[0], preferred_element_type=jnp.float32)
    out_ref[...] = y + b2_ref[0]


def _ffn(gid, pos0r, pos1r, x, W1, b1r, W2, b2r):
    grid_spec = pltpu.PrefetchScalarGridSpec(
        num_scalar_prefetch=1,
        grid=(NB,),
        in_specs=[
            pl.BlockSpec((1, T), lambda b, g: (0, 0)),
            pl.BlockSpec((1, T), lambda b, g: (0, 0)),
            pl.BlockSpec((T, D), lambda b, g: (0, 0)),
            pl.BlockSpec((1, D, F), lambda b, g: (g[b], 0, 0)),
            pl.BlockSpec((1, 1, F), lambda b, g: (g[b], 0, 0)),
            pl.BlockSpec((1, F, D), lambda b, g: (g[b], 0, 0)),
            pl.BlockSpec((1, 1, D), lambda b, g: (g[b], 0, 0)),
        ],
        out_specs=pl.BlockSpec((B, D), lambda b, g: (b, 0)),
    )
    return pl.pallas_call(
        _ffn_body,
        grid_spec=grid_spec,
        out_shape=jax.ShapeDtypeStruct((PADDED, D), jnp.float32),
    )(gid, pos0r, pos1r, x, W1, b1r, W2, b2r)


def _combine_body(pos_hbm, wp_hbm, ys_hbm, out_hbm,
                  idxE, idxO, wE, wO, bufE, bufO, sem):
    wid = lax.axis_index("s") * 2 + lax.axis_index("c")
    per = T // NW
    t0 = wid * per
    pltpu.sync_copy(pos_hbm.at[pl.ds(t0, per)], idxE)
    pltpu.sync_copy(pos_hbm.at[pl.ds(T + t0, per)], idxO)
    pltpu.sync_copy(wp_hbm.at[pl.ds(t0, per)], wE)
    pltpu.sync_copy(wp_hbm.at[pl.ds(T + t0, per)], wO)
    pltpu.async_copy(ys_hbm.at[idxE], bufE, sem).wait()
    pltpu.async_copy(ys_hbm.at[idxO], bufO, sem).wait()

    lane = lax.iota(jnp.int32, 16)

    def body(i, carry):
        g16 = pl.multiple_of((i // 16) * 16, 16)
        m = lane == (i & 15)
        we = jnp.sum(jnp.where(m, wE[pl.ds(g16, 16)], 0.0))
        wo = jnp.sum(jnp.where(m, wO[pl.ds(g16, 16)], 0.0))
        for d in range(D // 16):
            sl = pl.ds(16 * d, 16)
            bufE[i, sl] = we * bufE[i, sl] + wo * bufO[i, sl]
        return carry

    lax.fori_loop(0, per, body, 0)
    pltpu.sync_copy(bufE, out_hbm.at[pl.ds(t0, per)])


def _combine(pos, wp_flat, ys):
    mesh = plsc.VectorSubcoreMesh(core_axis_name="c", subcore_axis_name="s")
    fn = pl.kernel(
        _combine_body,
        out_type=jax.ShapeDtypeStruct((T, D), jnp.float32),
        mesh=mesh,
        compiler_params=pltpu.CompilerParams(needs_layout_passes=False),
        scratch_types=[
            pltpu.VMEM((T // NW,), jnp.int32),
            pltpu.VMEM((T // NW,), jnp.int32),
            pltpu.VMEM((T // NW,), jnp.float32),
            pltpu.VMEM((T // NW,), jnp.float32),
            pltpu.VMEM((T // NW, D), jnp.float32),
            pltpu.VMEM((T // NW, D), jnp.float32),
            pltpu.SemaphoreType.DMA,
        ],
    )
    return fn(pos, wp_flat, ys)


@jax.jit
def kernel(x, Wg, bg, W1, b1, W2, b2):
    wp, pos, gidv = _gate(x, Wg, bg.reshape(1, E))
    wp_flat = wp.T.reshape(NPAIR)
    pos_flat = pos.T.reshape(NPAIR)
    gid = gidv[:NB, 0]
    ys = _ffn(gid, pos[:, 0].reshape(1, T), pos[:, 1].reshape(1, T), x,
              W1, b1.reshape(E, 1, F), W2, b2.reshape(E, 1, D))
    return _combine(pos_flat, wp_flat, ys)


# speedup vs baseline: 1.1411x; 1.1411x over previous
import jax
import jax.numpy as jnp
from jax import lax
from jax.experimental import pallas as pl
from jax.experimental.pallas import tpu as pltpu
from jax.experimental.pallas import tpu_sc as plsc

D = 768
F = 3072
E = 8
T = 2048
K = 2
NPAIR = T * K
B = 128
NB = 40
PADDED = NB * B
NW = 32
CHUNK = 128


def _gate_body(x_ref, wg_ref, bg_ref, wp_ref, pos_ref, gid_ref):
    x = x_ref[...]
    scores = jnp.dot(x, wg_ref[...], preferred_element_type=jnp.float32)
    scores = scores + bg_ref[...]
    io = lax.broadcasted_iota(jnp.int32, (T, E), 1).astype(jnp.float32)
    m1 = jnp.max(scores, axis=1, keepdims=True)
    a1 = jnp.min(jnp.where(scores == m1, io, float(E)), axis=1, keepdims=True)
    masked = jnp.where(io == a1, -jnp.inf, scores)
    m2 = jnp.max(masked, axis=1, keepdims=True)
    a2 = jnp.min(jnp.where(masked == m2, io, float(E)), axis=1, keepdims=True)
    w1 = 1.0 / (1.0 + jnp.exp(m2 - m1))
    w2 = 1.0 - w1
    wp_ref[...] = jnp.concatenate([w1, w2], axis=1)

    oh1 = (io == a1).astype(jnp.float32)
    oh2 = (io == a2).astype(jnp.float32)
    tot1 = jnp.sum(oh1, axis=0, keepdims=True)
    tot = tot1 + jnp.sum(oh2, axis=0, keepdims=True)
    pu = jnp.floor((tot + float(B - 1)) * (1.0 / B)) * float(B)
    e0 = lax.broadcasted_iota(jnp.int32, (E, E), 0).astype(jnp.float32)
    e1 = lax.broadcasted_iota(jnp.int32, (E, E), 1).astype(jnp.float32)
    triE = (e0 < e1).astype(jnp.float32)
    base = jnp.dot(pu, triE, preferred_element_type=jnp.float32)

    t0 = lax.broadcasted_iota(jnp.int32, (T, T), 0)
    t1 = lax.broadcasted_iota(jnp.int32, (T, T), 1)
    triT = (t1 < t0).astype(jnp.bfloat16)
    oh_cat = jnp.concatenate([oh1, oh2], axis=1).astype(jnp.bfloat16)
    pref = jnp.dot(triT, oh_cat, preferred_element_type=jnp.float32)
    pos0 = jnp.sum(oh1 * (base + pref[:, :E]), axis=1, keepdims=True)
    pos1 = jnp.sum(oh2 * (base + tot1 + pref[:, E:]), axis=1, keepdims=True)
    pos_ref[...] = jnp.concatenate([pos0, pos1], axis=1).astype(jnp.int32)

    tp = jnp.sum(pu)
    bs = lax.broadcasted_iota(jnp.int32, (48, E), 0).astype(jnp.float32) * float(B)
    pend = base + pu
    gid = jnp.sum((bs >= pend).astype(jnp.float32), axis=1, keepdims=True)
    gid = jnp.minimum(gid, float(E - 1))
    valid = (bs < tp).astype(jnp.float32)
    gid_last = jnp.max(jnp.where(valid > 0.0, jnp.broadcast_to(gid, (48, E)),
                                 0.0))
    gidf = jnp.where(valid > 0.0, jnp.broadcast_to(gid, (48, E)), gid_last)
    gid_ref[...] = gidf.astype(jnp.int32)


def _gate(x, Wg, bg2):
    out_shapes = (
        jax.ShapeDtypeStruct((T, K), jnp.float32),
        jax.ShapeDtypeStruct((T, K), jnp.int32),
        jax.ShapeDtypeStruct((48, E), jnp.int32),
    )
    return pl.pallas_call(_gate_body, out_shape=out_shapes)(x, Wg, bg2)


def _dispatch_body(pos_hbm, x_hbm, xs_hbm, posbuf0, posbuf1, rows, sem):
    wid = lax.axis_index("s") * 2 + lax.axis_index("c")
    base_j = pl.multiple_of(wid * CHUNK, CHUNK)
    tok0 = pl.multiple_of(base_j & (T - 1), CHUNK)
    half = CHUNK // 2
    pltpu.sync_copy(pos_hbm.at[pl.ds(base_j, half)], posbuf0)
    pltpu.sync_copy(pos_hbm.at[pl.ds(base_j + half, half)], posbuf1)
    pltpu.sync_copy(x_hbm.at[pl.ds(tok0, half)], rows)
    pltpu.async_copy(rows, xs_hbm.at[posbuf0], sem).wait()
    pltpu.sync_copy(x_hbm.at[pl.ds(tok0 + half, half)], rows)
    pltpu.async_copy(rows, xs_hbm.at[posbuf1], sem).wait()


def _dispatch(pos_flat, x):
    mesh = plsc.VectorSubcoreMesh(core_axis_name="c", subcore_axis_name="s")
    fn = pl.kernel(
        _dispatch_body,
        out_type=jax.ShapeDtypeStruct((PADDED, D), jnp.float32),
        mesh=mesh,
        scratch_types=[
            pltpu.VMEM((CHUNK // 2,), jnp.int32),
            pltpu.VMEM((CHUNK // 2,), jnp.int32),
            pltpu.VMEM((CHUNK // 2, D), jnp.float32),
            pltpu.SemaphoreType.DMA,
        ],
    )
    return fn(pos_flat, x)


def _ffn_body(gid_ref, xs_ref, w1_ref, b1_ref, w2_ref, b2_ref, out_ref):
    h = jnp.dot(xs_ref[...], w1_ref[0], preferred_element_type=jnp.float32)
    h = jnp.maximum(h + b1_ref[0], 0.0)
    y = jnp.dot(h, w2_ref[0], preferred_element_type=jnp.float32)
    out_ref[...] = y + b2_ref[0]


def _ffn(gid, xs, W1, b1r, W2, b2r):
    grid_spec = pltpu.PrefetchScalarGridSpec(
        num_scalar_prefetch=1,
        grid=(NB,),
        in_specs=[
            pl.BlockSpec((B, D), lambda b, g: (b, 0)),
            pl.BlockSpec((1, D, F), lambda b, g: (g[b], 0, 0)),
            pl.BlockSpec((1, 1, F), lambda b, g: (g[b], 0, 0)),
            pl.BlockSpec((1, F, D), lambda b, g: (g[b], 0, 0)),
            pl.BlockSpec((1, 1, D), lambda b, g: (g[b], 0, 0)),
        ],
        out_specs=pl.BlockSpec((B, D), lambda b, g: (b, 0)),
    )
    return pl.pallas_call(
        _ffn_body,
        grid_spec=grid_spec,
        out_shape=jax.ShapeDtypeStruct((PADDED, D), jnp.float32),
    )(gid, xs, W1, b1r, W2, b2r)


def _combine_body(pos_hbm, wp_hbm, ys_hbm, out_hbm,
                  idxE, idxO, wE, wO, bufE, bufO, sem):
    wid = lax.axis_index("s") * 2 + lax.axis_index("c")
    per = T // NW
    t0 = wid * per
    pltpu.sync_copy(pos_hbm.at[pl.ds(t0, per)], idxE)
    pltpu.sync_copy(pos_hbm.at[pl.ds(T + t0, per)], idxO)
    pltpu.sync_copy(wp_hbm.at[pl.ds(t0, per)], wE)
    pltpu.sync_copy(wp_hbm.at[pl.ds(T + t0, per)], wO)
    pltpu.async_copy(ys_hbm.at[idxE], bufE, sem).wait()
    pltpu.async_copy(ys_hbm.at[idxO], bufO, sem).wait()

    lane = lax.iota(jnp.int32, 16)

    def body(i, carry):
        g16 = pl.multiple_of((i // 16) * 16, 16)
        m = lane == (i & 15)
        we = jnp.sum(jnp.where(m, wE[pl.ds(g16, 16)], 0.0))
        wo = jnp.sum(jnp.where(m, wO[pl.ds(g16, 16)], 0.0))
        for d in range(D // 16):
            sl = pl.ds(16 * d, 16)
            bufE[i, sl] = we * bufE[i, sl] + wo * bufO[i, sl]
        return carry

    lax.fori_loop(0, per, body, 0)
    pltpu.sync_copy(bufE, out_hbm.at[pl.ds(t0, per)])


def _combine(pos, wp_flat, ys):
    mesh = plsc.VectorSubcoreMesh(core_axis_name="c", subcore_axis_name="s")
    fn = pl.kernel(
        _combine_body,
        out_type=jax.ShapeDtypeStruct((T, D), jnp.float32),
        mesh=mesh,
        compiler_params=pltpu.CompilerParams(needs_layout_passes=False),
        scratch_types=[
            pltpu.VMEM((T // NW,), jnp.int32),
            pltpu.VMEM((T // NW,), jnp.int32),
            pltpu.VMEM((T // NW,), jnp.float32),
            pltpu.VMEM((T // NW,), jnp.float32),
            pltpu.VMEM((T // NW, D), jnp.float32),
            pltpu.VMEM((T // NW, D), jnp.float32),
            pltpu.SemaphoreType.DMA,
        ],
    )
    return fn(pos, wp_flat, ys)


@jax.jit
def kernel(x, Wg, bg, W1, b1, W2, b2):
    wp, pos, gidv = _gate(x, Wg, bg.reshape(1, E))
    wp_flat = wp.T.reshape(NPAIR)
    pos_flat = pos.T.reshape(NPAIR)
    gid = gidv[:NB, 0]
    xs = _dispatch(pos_flat, x)
    ys = _ffn(gid, xs, W1, b1.reshape(E, 1, F), W2, b2.reshape(E, 1, D))
    return _combine(pos_flat, wp_flat, ys)


# speedup vs baseline: 1.1463x; 1.0045x over previous
import jax
import jax.numpy as jnp
from jax import lax
from jax.experimental import pallas as pl
from jax.experimental.pallas import tpu as pltpu
from jax.experimental.pallas import tpu_sc as plsc

D = 768
F = 3072
E = 8
T = 2048
K = 2
NPAIR = T * K
B = 128
NB = 40
PADDED = NB * B
NW = 32
CHUNK = 128


def _gate_body(x_ref, wg_ref, bg_ref, wp_ref, pos_ref, gid_ref):
    x = x_ref[...]
    scores = jnp.dot(x, wg_ref[...], preferred_element_type=jnp.float32)
    scores = scores + bg_ref[...]
    io = lax.broadcasted_iota(jnp.int32, (T, E), 1).astype(jnp.float32)
    m1 = jnp.max(scores, axis=1, keepdims=True)
    a1 = jnp.min(jnp.where(scores == m1, io, float(E)), axis=1, keepdims=True)
    masked = jnp.where(io == a1, -jnp.inf, scores)
    m2 = jnp.max(masked, axis=1, keepdims=True)
    a2 = jnp.min(jnp.where(masked == m2, io, float(E)), axis=1, keepdims=True)
    w1 = 1.0 / (1.0 + jnp.exp(m2 - m1))
    w2 = 1.0 - w1
    wp_ref[...] = jnp.concatenate([w1, w2], axis=1)

    oh1 = (io == a1).astype(jnp.float32)
    oh2 = (io == a2).astype(jnp.float32)
    tot1 = jnp.sum(oh1, axis=0, keepdims=True)
    tot = tot1 + jnp.sum(oh2, axis=0, keepdims=True)
    pu = jnp.floor((tot + float(B - 1)) * (1.0 / B)) * float(B)
    e0 = lax.broadcasted_iota(jnp.int32, (E, E), 0).astype(jnp.float32)
    e1 = lax.broadcasted_iota(jnp.int32, (E, E), 1).astype(jnp.float32)
    triE = (e0 < e1).astype(jnp.float32)
    base = jnp.dot(pu, triE, preferred_element_type=jnp.float32)

    t0 = lax.broadcasted_iota(jnp.int32, (T, T), 0)
    t1 = lax.broadcasted_iota(jnp.int32, (T, T), 1)
    triT = (t1 < t0).astype(jnp.bfloat16)
    oh_cat = jnp.concatenate([oh1, oh2], axis=1).astype(jnp.bfloat16)
    pref = jnp.dot(triT, oh_cat, preferred_element_type=jnp.float32)
    pos0 = jnp.sum(oh1 * (base + pref[:, :E]), axis=1, keepdims=True)
    pos1 = jnp.sum(oh2 * (base + tot1 + pref[:, E:]), axis=1, keepdims=True)
    pos_ref[...] = jnp.concatenate([pos0, pos1], axis=1).astype(jnp.int32)

    tp = jnp.sum(pu)
    bs = lax.broadcasted_iota(jnp.int32, (48, E), 0).astype(jnp.float32) * float(B)
    pend = base + pu
    gid = jnp.sum((bs >= pend).astype(jnp.float32), axis=1, keepdims=True)
    gid = jnp.minimum(gid, float(E - 1))
    valid = (bs < tp).astype(jnp.float32)
    gid_last = jnp.max(jnp.where(valid > 0.0, jnp.broadcast_to(gid, (48, E)),
                                 0.0))
    gidf = jnp.where(valid > 0.0, jnp.broadcast_to(gid, (48, E)), gid_last)
    gid_ref[...] = gidf.astype(jnp.int32)


def _gate(x, Wg, bg2):
    out_shapes = (
        jax.ShapeDtypeStruct((T, K), jnp.float32),
        jax.ShapeDtypeStruct((T, K), jnp.int32),
        jax.ShapeDtypeStruct((48, E), jnp.int32),
    )
    return pl.pallas_call(_gate_body, out_shape=out_shapes)(x, Wg, bg2)


def _dispatch_body(pos_hbm, x_hbm, xs_hbm, posbuf0, posbuf1, rows, sem):
    wid = lax.axis_index("s") * 2 + lax.axis_index("c")
    base_j = pl.multiple_of(wid * CHUNK, CHUNK)
    tok0 = pl.multiple_of(base_j & (T - 1), CHUNK)
    half = CHUNK // 2
    pltpu.sync_copy(pos_hbm.at[pl.ds(base_j, half)], posbuf0)
    pltpu.sync_copy(pos_hbm.at[pl.ds(base_j + half, half)], posbuf1)
    pltpu.sync_copy(x_hbm.at[pl.ds(tok0, half)], rows)
    pltpu.async_copy(rows, xs_hbm.at[posbuf0], sem).wait()
    pltpu.sync_copy(x_hbm.at[pl.ds(tok0 + half, half)], rows)
    pltpu.async_copy(rows, xs_hbm.at[posbuf1], sem).wait()


def _dispatch(pos_flat, x):
    mesh = plsc.VectorSubcoreMesh(core_axis_name="c", subcore_axis_name="s")
    fn = pl.kernel(
        _dispatch_body,
        out_type=jax.ShapeDtypeStruct((PADDED, D), jnp.float32),
        mesh=mesh,
        scratch_types=[
            pltpu.VMEM((CHUNK // 2,), jnp.int32),
            pltpu.VMEM((CHUNK // 2,), jnp.int32),
            pltpu.VMEM((CHUNK // 2, D), jnp.float32),
            pltpu.SemaphoreType.DMA,
        ],
    )
    return fn(pos_flat, x)


def _ffn_body(gid_ref, xs_ref, w1_ref, b1_ref, w2_ref, b2_ref, out_ref):
    xb = xs_ref[...].astype(jnp.bfloat16)
    h = jnp.dot(xb, w1_ref[0].astype(jnp.bfloat16),
                preferred_element_type=jnp.float32)
    h = jnp.maximum(h + b1_ref[0], 0.0)
    y = jnp.dot(h.astype(jnp.bfloat16), w2_ref[0].astype(jnp.bfloat16),
                preferred_element_type=jnp.float32)
    out_ref[...] = y + b2_ref[0]


def _ffn(gid, xs, W1, b1r, W2, b2r):
    grid_spec = pltpu.PrefetchScalarGridSpec(
        num_scalar_prefetch=1,
        grid=(NB,),
        in_specs=[
            pl.BlockSpec((B, D), lambda b, g: (b, 0)),
            pl.BlockSpec((1, D, F), lambda b, g: (g[b], 0, 0)),
            pl.BlockSpec((1, 1, F), lambda b, g: (g[b], 0, 0)),
            pl.BlockSpec((1, F, D), lambda b, g: (g[b], 0, 0)),
            pl.BlockSpec((1, 1, D), lambda b, g: (g[b], 0, 0)),
        ],
        out_specs=pl.BlockSpec((B, D), lambda b, g: (b, 0)),
    )
    return pl.pallas_call(
        _ffn_body,
        grid_spec=grid_spec,
        out_shape=jax.ShapeDtypeStruct((PADDED, D), jnp.float32),
    )(gid, xs, W1, b1r, W2, b2r)


def _combine_body(pos_hbm, wp_hbm, ys_hbm, out_hbm,
                  idxE, idxO, wE, wO, bufE, bufO, sem):
    wid = lax.axis_index("s") * 2 + lax.axis_index("c")
    per = T // NW
    t0 = wid * per
    pltpu.sync_copy(pos_hbm.at[pl.ds(t0, per)], idxE)
    pltpu.sync_copy(pos_hbm.at[pl.ds(T + t0, per)], idxO)
    pltpu.sync_copy(wp_hbm.at[pl.ds(t0, per)], wE)
    pltpu.sync_copy(wp_hbm.at[pl.ds(T + t0, per)], wO)
    pltpu.async_copy(ys_hbm.at[idxE], bufE, sem).wait()
    pltpu.async_copy(ys_hbm.at[idxO], bufO, sem).wait()

    lane = lax.iota(jnp.int32, 16)

    def body(i, carry):
        g16 = pl.multiple_of((i // 16) * 16, 16)
        m = lane == (i & 15)
        we = jnp.sum(jnp.where(m, wE[pl.ds(g16, 16)], 0.0))
        wo = jnp.sum(jnp.where(m, wO[pl.ds(g16, 16)], 0.0))
        for d in range(D // 16):
            sl = pl.ds(16 * d, 16)
            bufE[i, sl] = we * bufE[i, sl] + wo * bufO[i, sl]
        return carry

    lax.fori_loop(0, per, body, 0)
    pltpu.sync_copy(bufE, out_hbm.at[pl.ds(t0, per)])


def _combine(pos, wp_flat, ys):
    mesh = plsc.VectorSubcoreMesh(core_axis_name="c", subcore_axis_name="s")
    fn = pl.kernel(
        _combine_body,
        out_type=jax.ShapeDtypeStruct((T, D), jnp.float32),
        mesh=mesh,
        compiler_params=pltpu.CompilerParams(needs_layout_passes=False),
        scratch_types=[
            pltpu.VMEM((T // NW,), jnp.int32),
            pltpu.VMEM((T // NW,), jnp.int32),
            pltpu.VMEM((T // NW,), jnp.float32),
            pltpu.VMEM((T // NW,), jnp.float32),
            pltpu.VMEM((T // NW, D), jnp.float32),
            pltpu.VMEM((T // NW, D), jnp.float32),
            pltpu.SemaphoreType.DMA,
        ],
    )
    return fn(pos, wp_flat, ys)


@jax.jit
def kernel(x, Wg, bg, W1, b1, W2, b2):
    wp, pos, gidv = _gate(x, Wg, bg.reshape(1, E))
    wp_flat = wp.T.reshape(NPAIR)
    pos_flat = pos.T.reshape(NPAIR)
    gid = gidv[:NB, 0]
    xs = _dispatch(pos_flat, x)
    ys = _ffn(gid, xs, W1, b1.reshape(E, 1, F), W2, b2.reshape(E, 1, D))
    return _combine(pos_flat, wp_flat, ys)
